# baseline (speedup 1.0000x reference)
import functools

import jax
import jax.numpy as jnp
from jax import lax
from jax.experimental import pallas as pl
from jax.experimental.pallas import tpu as pltpu
from jax.experimental.pallas import tpu_sc as plsc

N = 100000
E = 3200000
DE = 16
DH = 128

NC = 2
NS = 16

TRN, SLN, LNN = 2, 8, 128
TCN = E // LNN
HALF_TC = TCN // NC
CH = 50
NCH = HALF_TC // CH


def _sc_segment_sum(ev, receivers):
    mesh = plsc.VectorSubcoreMesh(core_axis_name="c", subcore_axis_name="s")

    @functools.partial(
        pl.kernel,
        mesh=mesh,
        out_type=jax.ShapeDtypeStruct((NC, NS, N), jnp.float32),
        scratch_types=[
            pltpu.VMEM((N,), jnp.float32),
            pltpu.VMEM((CH, LNN), jnp.float32),
            pltpu.VMEM((CH, LNN), jnp.float32),
            pltpu.VMEM((CH * LNN,), jnp.int32),
            pltpu.VMEM((CH * LNN,), jnp.int32),
            pltpu.SemaphoreType.DMA,
            pltpu.SemaphoreType.DMA,
            pltpu.SemaphoreType.DMA,
            pltpu.SemaphoreType.DMA,
        ],
        compiler_params=pltpu.CompilerParams(
            use_tc_tiling_on_sc=False, needs_layout_passes=False,
        ),
    )
    def k(ev_hbm, recv_hbm, out_hbm, acc_v,
          vals0, vals1, idx0, idx1, sv0, sv1, si0, si1):
        c = lax.axis_index("c")
        s = lax.axis_index("s")
        tr = s // SLN
        sl = s % SLN

        zero = jnp.zeros((16,), jnp.float32)
        tc_base = c * HALF_TC
        vb = (vals0, vals1)
        ib = (idx0, idx1)
        sv = (sv0, sv1)
        si = (si0, si1)

        def refs(k_, slot):
            tcs = tc_base + k_ * CH
            return (
                (ev_hbm.at[tr, pl.ds(tcs, CH), sl], vb[slot], sv[slot]),
                (recv_hbm.at[pl.ds(tcs * LNN, CH * LNN)], ib[slot], si[slot]),
            )

        def issue(k_, slot):
            for args in refs(k_, slot):
                pltpu.async_copy(*args)

        def wait(k_, slot):
            for args in refs(k_, slot):
                pltpu.make_async_copy(*args).wait()

        def compute(slot):
            @pl.loop(0, CH)
            def _(j):
                pairs = [
                    (
                        ib[slot][pl.ds(j * LNN + jj * 16, 16)],
                        vb[slot][j, pl.ds(jj * 16, 16)],
                    )
                    for jj in range(SLN)
                ]
                for idx16, val16 in pairs:
                    plsc.addupdate_scatter(acc_v, [idx16], val16)

        issue(0, 0)

        @pl.loop(0, N // 16)
        def _(i):
            acc_v[pl.ds(i * 16, 16)] = zero

        @pl.loop(0, NCH // 2)
        def _(p):
            k0 = 2 * p

            @pl.when(k0 + 1 < NCH)
            def _():
                issue(k0 + 1, 1)

            wait(k0, 0)
            compute(0)

            @pl.when(k0 + 2 < NCH)
            def _():
                issue(k0 + 2, 0)

            wait(k0 + 1, 1)
            compute(1)

        pltpu.sync_copy(acc_v, out_hbm.at[c, s])

    return k(ev, receivers)


_BLK = 4096


def _hg_body(nodes_ref, whh_ref, out_ref):
    out_ref[...] = lax.dot_general(
        nodes_ref[...].astype(jnp.bfloat16), whh_ref[...].astype(jnp.bfloat16),
        (((1,), (1,)), ((), ())),
        preferred_element_type=jnp.float32,
    ).astype(jnp.bfloat16)


def _tc_hg(nodes, W_hh):
    return pl.pallas_call(
        _hg_body,
        grid=(pl.cdiv(N, _BLK),),
        in_specs=[
            pl.BlockSpec((_BLK, DH), lambda i: (i, 0)),
            pl.BlockSpec((3 * DH, DH), lambda i: (0, 0)),
        ],
        out_specs=pl.BlockSpec((_BLK, 3 * DH), lambda i: (i, 0)),
        out_shape=jax.ShapeDtypeStruct((N, 3 * DH), jnp.bfloat16),
    )(nodes, W_hh)


def _gru_body(p_ref, nodes_ref, hg_ref, wih_ref, b_ref, bn_ref, out_ref):
    aggr_t = p_ref[0] + p_ref[1]
    h = nodes_ref[...]
    ig = lax.dot_general(
        aggr_t.astype(jnp.bfloat16), wih_ref[...].astype(jnp.bfloat16),
        (((0,), (1,)), ((), ())),
        preferred_element_type=jnp.float32,
    ) + b_ref[...]
    hg = hg_ref[...].astype(jnp.float32)
    ir, iz, inew = ig[:, :DH], ig[:, DH:2 * DH], ig[:, 2 * DH:]
    hr, hz, hn = hg[:, :DH], hg[:, DH:2 * DH], hg[:, 2 * DH:]
    reset = 0.5 * jnp.tanh(0.5 * (ir + hr)) + 0.5
    inp = 0.5 * jnp.tanh(0.5 * (iz + hz)) + 0.5
    new = jnp.tanh(inew + reset * (hn + bn_ref[...]))
    out_ref[...] = new + inp * (h - new)


def _tc_gru(partials, nodes, hg, W_ih, b2, bn2):
    grid = (pl.cdiv(N, _BLK),)
    return pl.pallas_call(
        _gru_body,
        grid=grid,
        in_specs=[
            pl.BlockSpec((NC, NS, _BLK), lambda i: (0, 0, i)),
            pl.BlockSpec((_BLK, DH), lambda i: (i, 0)),
            pl.BlockSpec((_BLK, 3 * DH), lambda i: (i, 0)),
            pl.BlockSpec((3 * DH, DE), lambda i: (0, 0)),
            pl.BlockSpec((1, 3 * DH), lambda i: (0, 0)),
            pl.BlockSpec((1, DH), lambda i: (0, 0)),
        ],
        out_specs=pl.BlockSpec((_BLK, DH), lambda i: (i, 0)),
        out_shape=jax.ShapeDtypeStruct((N, DH), jnp.float32),
    )(partials, nodes, hg, W_ih, b2, bn2)


def kernel(nodes, edges, receivers, senders, W_ih, W_hh, b, b_n):
    del senders
    ev = (
        jnp.transpose(edges)
        .reshape(TRN, SLN, TCN, LNN)
        .transpose(0, 2, 1, 3)
    )
    partials = _sc_segment_sum(ev, receivers)
    hg = _tc_hg(nodes, W_hh)
    return _tc_gru(
        partials, nodes, hg, W_ih,
        b.reshape(1, 3 * DH), b_n.reshape(1, DH),
    )


# speedup vs baseline: 1.0455x; 1.0455x over previous
import functools

import jax
import jax.numpy as jnp
from jax import lax
from jax.experimental import pallas as pl
from jax.experimental.pallas import tpu as pltpu
from jax.experimental.pallas import tpu_sc as plsc

N = 100000
E = 3200000
DE = 16
DH = 128

NC = 2
NS = 16

TRN, SLN, LNN = 2, 8, 128
TCN = E // LNN
HALF_TC = TCN // NC
CH = 50
NCH = HALF_TC // CH


def _sc_segment_sum(ev, receivers):
    mesh = plsc.VectorSubcoreMesh(core_axis_name="c", subcore_axis_name="s")

    @functools.partial(
        pl.kernel,
        mesh=mesh,
        out_type=jax.ShapeDtypeStruct((NC, NS, N), jnp.float32),
        scratch_types=[
            pltpu.VMEM((N,), jnp.float32),
            pltpu.VMEM((CH, LNN), jnp.float32),
            pltpu.VMEM((CH, LNN), jnp.float32),
            pltpu.VMEM((CH * LNN,), jnp.int32),
            pltpu.VMEM((CH * LNN,), jnp.int32),
            pltpu.SemaphoreType.DMA,
            pltpu.SemaphoreType.DMA,
            pltpu.SemaphoreType.DMA,
            pltpu.SemaphoreType.DMA,
        ],
        compiler_params=pltpu.CompilerParams(
            use_tc_tiling_on_sc=False, needs_layout_passes=False,
        ),
    )
    def k(ev_hbm, recv_hbm, out_hbm, acc_v,
          vals0, vals1, idx0, idx1, sv0, sv1, si0, si1):
        c = lax.axis_index("c")
        s = lax.axis_index("s")
        tr = s // SLN
        sl = s % SLN

        zero = jnp.zeros((16,), jnp.float32)
        tc_base = c * HALF_TC
        vb = (vals0, vals1)
        ib = (idx0, idx1)
        sv = (sv0, sv1)
        si = (si0, si1)

        def refs(k_, slot):
            tcs = tc_base + k_ * CH
            return (
                (ev_hbm.at[tr, pl.ds(tcs, CH), sl], vb[slot], sv[slot]),
                (recv_hbm.at[pl.ds(tcs * LNN, CH * LNN)], ib[slot], si[slot]),
            )

        def issue(k_, slot):
            for args in refs(k_, slot):
                pltpu.async_copy(*args)

        def wait(k_, slot):
            for args in refs(k_, slot):
                pltpu.make_async_copy(*args).wait()

        def compute(slot):
            @pl.loop(0, CH)
            def _(j):
                pairs = [
                    (
                        ib[slot][pl.ds(j * LNN + jj * 16, 16)],
                        vb[slot][j, pl.ds(jj * 16, 16)],
                    )
                    for jj in range(SLN)
                ]
                for idx16, val16 in pairs:
                    plsc.addupdate_scatter(acc_v, [idx16], val16)

        issue(0, 0)

        @pl.loop(0, N // 16)
        def _(i):
            acc_v[pl.ds(i * 16, 16)] = zero

        @pl.loop(0, NCH // 2)
        def _(p):
            k0 = 2 * p

            @pl.when(k0 + 1 < NCH)
            def _():
                issue(k0 + 1, 1)

            wait(k0, 0)
            compute(0)

            @pl.when(k0 + 2 < NCH)
            def _():
                issue(k0 + 2, 0)

            wait(k0 + 1, 1)
            compute(1)

        pltpu.sync_copy(acc_v, out_hbm.at[c, s])

    return k(ev, receivers)


_BLK = 4096


def _gru_body(p_ref, nodes_ref, wih_ref, whh_ref, b_ref, bn_ref, out_ref):
    aggr_t = p_ref[0] + p_ref[1]
    h = nodes_ref[...]
    ig = lax.dot_general(
        aggr_t.astype(jnp.bfloat16), wih_ref[...].astype(jnp.bfloat16),
        (((0,), (1,)), ((), ())),
        preferred_element_type=jnp.float32,
    ) + b_ref[...]
    hg = lax.dot_general(
        h.astype(jnp.bfloat16), whh_ref[...].astype(jnp.bfloat16),
        (((1,), (1,)), ((), ())),
        preferred_element_type=jnp.float32,
    )
    ir, iz, inew = ig[:, :DH], ig[:, DH:2 * DH], ig[:, 2 * DH:]
    hr, hz, hn = hg[:, :DH], hg[:, DH:2 * DH], hg[:, 2 * DH:]
    reset = 0.5 * jnp.tanh(0.5 * (ir + hr)) + 0.5
    inp = 0.5 * jnp.tanh(0.5 * (iz + hz)) + 0.5
    new = jnp.tanh(inew + reset * (hn + bn_ref[...]))
    out_ref[...] = new + inp * (h - new)


def _tc_gru(partials, nodes, W_ih, W_hh, b2, bn2):
    grid = (pl.cdiv(N, _BLK),)
    return pl.pallas_call(
        _gru_body,
        grid=grid,
        in_specs=[
            pl.BlockSpec((NC, NS, _BLK), lambda i: (0, 0, i)),
            pl.BlockSpec((_BLK, DH), lambda i: (i, 0)),
            pl.BlockSpec((3 * DH, DE), lambda i: (0, 0)),
            pl.BlockSpec((3 * DH, DH), lambda i: (0, 0)),
            pl.BlockSpec((1, 3 * DH), lambda i: (0, 0)),
            pl.BlockSpec((1, DH), lambda i: (0, 0)),
        ],
        out_specs=pl.BlockSpec((_BLK, DH), lambda i: (i, 0)),
        out_shape=jax.ShapeDtypeStruct((N, DH), jnp.float32),
    )(partials, nodes, W_ih, W_hh, b2, bn2)


def kernel(nodes, edges, receivers, senders, W_ih, W_hh, b, b_n):
    del senders
    ev = (
        jnp.transpose(edges)
        .reshape(TRN, SLN, TCN, LNN)
        .transpose(0, 2, 1, 3)
    )
    partials = _sc_segment_sum(ev, receivers)
    return _tc_gru(
        partials, nodes, W_ih, W_hh,
        b.reshape(1, 3 * DH), b_n.reshape(1, DH),
    )
